# baseline (speedup 1.0000x reference)
import jax
import jax.numpy as jnp
from jax.experimental import pallas as pl

_EXER_N, _KN_N, _STU_N, _EMB = 50000, 10000, 50000, 128


def _rup(n, m):
    return (n + m - 1) // m * m


def _proj_body(h_ref, wfc_ref, wab_ref, z_ref, ab_ref):
    z = h_ref[...] @ wfc_ref[...]
    z_ref[...] = z
    ab_ref[...] = z @ wab_ref[...]


def _proj(h, w_fc, w_attn):
    n = h.shape[0]
    tn = 512
    n_p = _rup(n, tn)
    hp = jnp.pad(h, ((0, n_p - n), (0, 0)))
    w_ab = jnp.concatenate([w_attn[:_EMB], w_attn[_EMB:]], axis=1)
    z, ab = pl.pallas_call(
        _proj_body,
        grid=(n_p // tn,),
        in_specs=[
            pl.BlockSpec((tn, _EMB), lambda i: (i, 0)),
            pl.BlockSpec((_EMB, _EMB), lambda i: (0, 0)),
            pl.BlockSpec((_EMB, 2), lambda i: (0, 0)),
        ],
        out_specs=[
            pl.BlockSpec((tn, _EMB), lambda i: (i, 0)),
            pl.BlockSpec((tn, 2), lambda i: (i, 0)),
        ],
        out_shape=[
            jax.ShapeDtypeStruct((n_p, _EMB), jnp.float32),
            jax.ShapeDtypeStruct((n_p, 2), jnp.float32),
        ],
    )(hp, w_fc, w_ab)
    return z, ab


def _exp_body(x_ref, y_ref, o_ref):
    o_ref[...] = jnp.exp(x_ref[...] + y_ref[...])


def _edge_exp(a_src, b_dst):
    e = a_src.shape[0]
    blk = 256 * 128
    e_p = _rup(e, blk)
    x = jnp.pad(a_src, (0, e_p - e)).reshape(e_p // 128, 128)
    y = jnp.pad(b_dst, (0, e_p - e)).reshape(e_p // 128, 128)
    ex = pl.pallas_call(
        _exp_body,
        grid=(e_p // blk,),
        in_specs=[
            pl.BlockSpec((256, 128), lambda i: (i, 0)),
            pl.BlockSpec((256, 128), lambda i: (i, 0)),
        ],
        out_specs=pl.BlockSpec((256, 128), lambda i: (i, 0)),
        out_shape=jax.ShapeDtypeStruct((e_p // 128, 128), jnp.float32),
    )(x, y)
    return ex.reshape(e_p)[:e]


def _msg_body(z_ref, ex_ref, s_ref, o_ref):
    o_ref[...] = z_ref[...] * (ex_ref[...] / jnp.maximum(s_ref[...], 1e-9))


def _edge_msg(z_src, ex, s_dst):
    e = z_src.shape[0]
    te = 512
    e_p = _rup(e, te)
    zp = jnp.pad(z_src, ((0, e_p - e), (0, 0)))
    exp_ = jnp.pad(ex, (0, e_p - e)).reshape(e_p, 1)
    sp = jnp.pad(s_dst, (0, e_p - e), constant_values=1.0).reshape(e_p, 1)
    msg = pl.pallas_call(
        _msg_body,
        grid=(e_p // te,),
        in_specs=[
            pl.BlockSpec((te, _EMB), lambda i: (i, 0)),
            pl.BlockSpec((te, 1), lambda i: (i, 0)),
            pl.BlockSpec((te, 1), lambda i: (i, 0)),
        ],
        out_specs=pl.BlockSpec((te, _EMB), lambda i: (i, 0)),
        out_shape=jax.ShapeDtypeStruct((e_p, _EMB), jnp.float32),
    )(zp, exp_, sp)
    return msg[:e]


def _layer(h, src, dst, num_nodes, w_fc, w_attn):
    z, ab = _proj(h, w_fc, w_attn)
    a_src = ab[src, 0]
    b_dst = ab[dst, 1]
    ex = _edge_exp(a_src, b_dst)
    s = jax.ops.segment_sum(ex, dst, num_segments=num_nodes)
    msg = _edge_msg(z[src], ex, s[dst])
    return jax.ops.segment_sum(msg, dst, num_segments=num_nodes)


def _fuse_body(a_ref, x_ref, y_ref, w1a_ref, w1b_ref, w2a_ref, w2b_ref,
               b1_ref, b2_ref, o_ref):
    a = a_ref[...]
    x = x_ref[...]
    y = y_ref[...]
    s1 = a @ w1a_ref[...] + x @ w1b_ref[...] + b1_ref[...]
    s2 = a @ w2a_ref[...] + y @ w2b_ref[...] + b2_ref[...]
    m = jnp.maximum(s1, s2)
    e1 = jnp.exp(s1 - m)
    e2 = jnp.exp(s2 - m)
    d = e1 + e2
    o_ref[...] = a + (e1 / d) * x + (e2 / d) * y


def _fuse(a, x, y, w1, b1, w2, b2):
    n = a.shape[0]
    tn = 512
    n_p = _rup(n, tn)
    pad = ((0, n_p - n), (0, 0))
    out = pl.pallas_call(
        _fuse_body,
        grid=(n_p // tn,),
        in_specs=[
            pl.BlockSpec((tn, _EMB), lambda i: (i, 0)),
            pl.BlockSpec((tn, _EMB), lambda i: (i, 0)),
            pl.BlockSpec((tn, _EMB), lambda i: (i, 0)),
            pl.BlockSpec((_EMB, 1), lambda i: (0, 0)),
            pl.BlockSpec((_EMB, 1), lambda i: (0, 0)),
            pl.BlockSpec((_EMB, 1), lambda i: (0, 0)),
            pl.BlockSpec((_EMB, 1), lambda i: (0, 0)),
            pl.BlockSpec((1, 1), lambda i: (0, 0)),
            pl.BlockSpec((1, 1), lambda i: (0, 0)),
        ],
        out_specs=pl.BlockSpec((tn, _EMB), lambda i: (i, 0)),
        out_shape=jax.ShapeDtypeStruct((n_p, _EMB), jnp.float32),
    )(jnp.pad(a, pad), jnp.pad(x, pad), jnp.pad(y, pad),
      w1[:_EMB], w1[_EMB:], w2[:_EMB], w2[_EMB:],
      b1.reshape(1, 1), b2.reshape(1, 1))
    return out[:n]


def _add_body(x_ref, y_ref, o_ref):
    o_ref[...] = x_ref[...] + y_ref[...]


def _add(x, y):
    n = x.shape[0]
    tn = 512
    n_p = _rup(n, tn)
    pad = ((0, n_p - n), (0, 0))
    out = pl.pallas_call(
        _add_body,
        grid=(n_p // tn,),
        in_specs=[
            pl.BlockSpec((tn, _EMB), lambda i: (i, 0)),
            pl.BlockSpec((tn, _EMB), lambda i: (i, 0)),
        ],
        out_specs=pl.BlockSpec((tn, _EMB), lambda i: (i, 0)),
        out_shape=jax.ShapeDtypeStruct((n_p, _EMB), jnp.float32),
    )(jnp.pad(x, pad), jnp.pad(y, pad))
    return out[:n]


def kernel(kn_emb, exer_emb, all_stu_emb, und_edges, ek_edges, ke_edges,
           eu_edges, ue_edges, W_und_fc, W_und_attn, W_ek_fc, W_ek_attn,
           W_ke_fc, W_ke_attn, W_eu_fc, W_eu_attn, W_ue_fc, W_ue_attn,
           W_k2, b_k2, W_k3, b_k3, W_e1, b_e1, W_e2, b_e2):
    k_und = _layer(kn_emb, und_edges[0], und_edges[1], _KN_N,
                   W_und_fc, W_und_attn)
    e_k = jnp.concatenate([exer_emb, kn_emb], axis=0)
    e_to_k = _layer(e_k, ek_edges[0], ek_edges[1], _EXER_N + _KN_N,
                    W_ek_fc, W_ek_attn)
    k_to_e = _layer(e_k, ke_edges[0], ke_edges[1], _EXER_N + _KN_N,
                    W_ke_fc, W_ke_attn)
    e_u = jnp.concatenate([exer_emb, all_stu_emb], axis=0)
    e_to_u = _layer(e_u, eu_edges[0], eu_edges[1], _EXER_N + _STU_N,
                    W_eu_fc, W_eu_attn)
    u_to_e = _layer(e_u, ue_edges[0], ue_edges[1], _EXER_N + _STU_N,
                    W_ue_fc, W_ue_attn)

    kn_out = _fuse(kn_emb, k_und, e_to_k[_EXER_N:], W_k2, b_k2, W_k3, b_k3)
    exer_out = _fuse(exer_emb, k_to_e[:_EXER_N], u_to_e[:_EXER_N],
                     W_e1, b_e1, W_e2, b_e2)
    stu_out = _add(all_stu_emb, e_to_u[_EXER_N:])
    return (kn_out, exer_out, stu_out)
